# Initial kernel scaffold; baseline (speedup 1.0000x reference)
import jax
import jax.numpy as jnp
from jax.experimental import pallas as pl


def kernel(x, pos, pseudo0, pseudo1, pseudo2, pseudo3, pseudo4, pseudo5, params, edge_index0, edge_index1, edge_index2, edge_index3, edge_index4, edge_index5, cluster1, cluster2, cluster3, cluster4, cluster5):
    raise NotImplementedError("write your pallas kernel here")


# speedup vs baseline: 1.1444x; 1.1444x over previous
import jax
import jax.numpy as jnp
from jax.experimental import pallas as pl

_N_LVL = [100000, 25000, 6250, 1600, 400, 100]


def _basis_kernel(pt_ref, out_ref):
    p = pt_ref[...]
    b = (1.0 - jnp.abs(p[0] * 2.0 - 1.0)) * (1.0 - jnp.abs(p[1] * 2.0 - 1.0)) * (1.0 - jnp.abs(p[2] * 2.0 - 1.0))
    out_ref[...] = b[None, :]


def _basis(pseudo):
    e = pseudo.shape[0]
    blk = min(e, 131072)
    grid = e // blk
    pt = pseudo.T
    out = pl.pallas_call(
        _basis_kernel,
        grid=(grid,),
        in_specs=[pl.BlockSpec((3, blk), lambda i: (0, i))],
        out_specs=pl.BlockSpec((1, blk), lambda i: (0, i)),
        out_shape=jax.ShapeDtypeStruct((1, e), jnp.float32),
    )(pt)
    return out[0]


def _spline_conv(p, x, ei, pseudo):
    src, dst = ei[0], ei[1]
    basis = _basis(pseudo)
    n = x.shape[0]
    g = jax.ops.segment_sum(jnp.take(x, src, axis=0) * basis[:, None], dst, num_segments=n)
    agg = g @ p['Wm']
    deg = jax.ops.segment_sum(jnp.ones((ei.shape[1],), jnp.float32), dst, num_segments=n)
    return agg / jnp.clip(deg, 1.0)[:, None] + x @ p['Wr'] + p['b']


def _lin(p, x):
    return x @ p['W'] + p['b']


def _pool_max(x, cluster, n_out):
    out = jax.ops.segment_max(x, cluster, num_segments=n_out)
    return jnp.where(jnp.isfinite(out), out, 0.0)


def _pool_mean(x, cluster, n_out):
    s = jax.ops.segment_sum(x, cluster, num_segments=n_out)
    c = jax.ops.segment_sum(jnp.ones((x.shape[0],), jnp.float32), cluster, num_segments=n_out)
    return s / jnp.clip(c, 1.0)[:, None]


def kernel(x, pos, pseudo0, pseudo1, pseudo2, pseudo3, pseudo4, pseudo5, params, edge_index0, edge_index1, edge_index2, edge_index3, edge_index4, edge_index5, cluster1, cluster2, cluster3, cluster4, cluster5):
    pseudos = [pseudo0, pseudo1, pseudo2, pseudo3, pseudo4, pseudo5]
    edges = [edge_index0, edge_index1, edge_index2, edge_index3, edge_index4, edge_index5]
    clusters = [cluster1, cluster2, cluster3, cluster4, cluster5]
    elu = jax.nn.elu
    x0 = elu(_spline_conv(params['conv1'], x, edges[0], pseudos[0]))
    x1p = _pool_max(x0, clusters[0], _N_LVL[1])
    h = jnp.concatenate([x1p, jnp.ones((_N_LVL[1], 1), jnp.float32)], axis=1)
    h = elu(_spline_conv(params['conv2'], h, edges[1], pseudos[1]))
    h = _spline_conv(params['conv22'], h, edges[1], pseudos[1])
    x1 = elu(h + _lin(params['skip1'], x1p))
    x2p = _pool_max(x1, clusters[1], _N_LVL[2])
    h = jnp.concatenate([x2p, jnp.ones((_N_LVL[2], 1), jnp.float32)], axis=1)
    h = elu(_spline_conv(params['conv3'], h, edges[2], pseudos[2]))
    h = _spline_conv(params['conv32'], h, edges[2], pseudos[2])
    x2 = elu(h + x2p)
    x3p = _pool_max(x2, clusters[2], _N_LVL[3])
    h = jnp.concatenate([x3p, jnp.ones((_N_LVL[3], 1), jnp.float32)], axis=1)
    h = elu(_spline_conv(params['conv4'], h, edges[3], pseudos[3]))
    h = _spline_conv(params['conv42'], h, edges[3], pseudos[3])
    x3 = elu(h + x3p)
    x4p = _pool_max(x3, clusters[3], _N_LVL[4])
    h = jnp.concatenate([x4p, jnp.ones((_N_LVL[4], 1), jnp.float32)], axis=1)
    h = elu(_spline_conv(params['conv5'], h, edges[4], pseudos[4]))
    h = _spline_conv(params['conv52'], h, edges[4], pseudos[4])
    x4 = elu(h + _lin(params['skip2'], x4p))
    x5p = _pool_max(x4, clusters[4], _N_LVL[5])
    h = jnp.concatenate([x5p, jnp.ones((_N_LVL[5], 1), jnp.float32)], axis=1)
    h = elu(_spline_conv(params['conv6'], h, edges[5], pseudos[5]))
    h = _spline_conv(params['conv62'], h, edges[5], pseudos[5])
    x5 = elu(h + _lin(params['skip3'], x5p))
    x5 = _lin(params['fc1'], x5)
    up = jnp.take(jnp.take(x5, clusters[4], axis=0), clusters[3], axis=0)
    cat = jnp.concatenate([up, jnp.take(x4, clusters[3], axis=0), _lin(params['skip_out'], x3)], axis=1)
    r = elu(_spline_conv(params['convRPN1'], cat, edges[3], pseudos[3]))
    r = elu(_spline_conv(params['convRPN2'], r, edges[3], pseudos[3]))
    r = _spline_conv(params['convRPN3'], r, edges[3], pseudos[3])
    pos_c = pos
    for l in range(1, 6):
        pos_c = _pool_mean(pos_c, [None, *[clusters[i] for i in range(5)]][l], _N_LVL[l])
    return (jax.nn.log_softmax(r[:, :2], axis=1), r[:, 2:], pos_c)


# speedup vs baseline: 4.9109x; 4.2914x over previous
import functools

import jax
import jax.numpy as jnp
from jax import lax
from jax.experimental import pallas as pl
from jax.experimental.pallas import tpu as pltpu
from jax.experimental.pallas import tpu_sc as plsc

_N_LVL = [100000, 25000, 6250, 1600, 400, 100]

_NC = 2
_NS = 16
_B = 128


def _ceil_to(x, m):
    return (x + m - 1) // m * m


def _basis_body(pt_ref, out_ref):
    p = pt_ref[...]
    b = (1.0 - jnp.abs(p[0] * 2.0 - 1.0)) * (1.0 - jnp.abs(p[1] * 2.0 - 1.0)) * (1.0 - jnp.abs(p[2] * 2.0 - 1.0))
    out_ref[...] = b[None, :]


def _basis(pseudo, e_pad):
    e = pseudo.shape[0]
    pt = jnp.pad(pseudo, ((0, e_pad - e), (0, 0))).T
    blk = 4096
    out = pl.pallas_call(
        _basis_body,
        grid=(e_pad // blk,),
        in_specs=[pl.BlockSpec((3, blk), lambda i: (0, i))],
        out_specs=pl.BlockSpec((1, blk), lambda i: (0, i)),
        out_shape=jax.ShapeDtypeStruct((1, e_pad), jnp.float32),
    )(pt)
    return out[0]


@functools.lru_cache(maxsize=None)
def _make_edge_agg(n_in, w, e_pad, n_pad):
    chunks = e_pad // (_NC * _NS * _B)
    rz = n_pad // _NS
    nslice = w // 16
    mesh = plsc.VectorSubcoreMesh(core_axis_name="c", subcore_axis_name="s")

    @functools.partial(
        pl.kernel,
        out_type=jax.ShapeDtypeStruct((_NC, n_pad, w), jnp.float32),
        mesh=mesh,
        scratch_types=[
            pltpu.VMEM((_B,), jnp.int32),
            pltpu.VMEM((_B,), jnp.int32),
            pltpu.VMEM((_B,), jnp.float32),
            pltpu.VMEM((_B, w), jnp.float32),
            pltpu.VMEM_SHARED((n_pad, w), jnp.float32),
            pltpu.SemaphoreType.DMA,
        ],
        compiler_params=pltpu.CompilerParams(use_tc_tiling_on_sc=False),
    )
    def k(x_hbm, src_hbm, dst_hbm, bas_hbm, zro_hbm, out_hbm,
          src_v, dst_v, bas_v, rows_v, acc, sem):
        c = lax.axis_index("c")
        s = lax.axis_index("s")
        wid = s * _NC + c
        pltpu.sync_copy(zro_hbm.at[pl.ds(s * rz, rz)], acc.at[pl.ds(s * rz, rz)])
        plsc.subcore_barrier()

        def chunk_body(kk, _):
            base = (wid * chunks + kk) * _B
            pltpu.sync_copy(src_hbm.at[pl.ds(base, _B)], src_v)
            pltpu.sync_copy(dst_hbm.at[pl.ds(base, _B)], dst_v)
            pltpu.sync_copy(bas_hbm.at[pl.ds(base, _B)], bas_v)
            pltpu.async_copy(x_hbm.at[src_v], rows_v, sem).wait()

            def grp_body(t, _):
                bvec = bas_v[pl.ds(16 * t, 16)]
                for i in range(16):
                    b = bvec[i]
                    r = 16 * t + i
                    for j in range(nslice):
                        rows_v[r, pl.ds(16 * j, 16)] = rows_v[r, pl.ds(16 * j, 16)] * b
                return 0

            lax.fori_loop(0, _B // 16, grp_body, 0)
            pltpu.sync_copy(rows_v, acc.at[dst_v], add=True)
            return 0

        lax.fori_loop(0, chunks, chunk_body, 0)
        plsc.subcore_barrier()
        pltpu.sync_copy(acc.at[pl.ds(s * rz, rz)], out_hbm.at[c, pl.ds(s * rz, rz)])

    return k


def _edge_agg(x_pad, src_p, dst_p, bas, n_out):
    n_in, w = x_pad.shape
    e_pad = bas.shape[0]
    n_pad = _ceil_to(n_out, 128)
    k = _make_edge_agg(n_in, w, e_pad, n_pad)
    zro = jnp.zeros((n_pad, w), jnp.float32)
    out = k(x_pad, src_p, dst_p, bas, zro)
    return (out[0] + out[1])[:n_out]


@functools.lru_cache(maxsize=None)
def _make_deg_s(e, e_pad, n_pad):
    chunks = e_pad // (_NC * _NS * _B)
    rz = n_pad // _NS
    mesh = plsc.VectorSubcoreMesh(core_axis_name="c", subcore_axis_name="s")

    @functools.partial(
        pl.kernel,
        out_type=jax.ShapeDtypeStruct((_NC, n_pad, 16), jnp.float32),
        mesh=mesh,
        scratch_types=[
            pltpu.VMEM((_B,), jnp.int32),
            pltpu.VMEM((_B,), jnp.float32),
            pltpu.VMEM((_B, 16), jnp.float32),
            pltpu.VMEM_SHARED((n_pad, 16), jnp.float32),
        ],
        compiler_params=pltpu.CompilerParams(use_tc_tiling_on_sc=False),
    )
    def k(dst_hbm, bas_hbm, zro_hbm, out_hbm, dst_v, bas_v, rows_v, acc):
        c = lax.axis_index("c")
        s = lax.axis_index("s")
        wid = s * _NC + c
        e_deg = jnp.where(lax.iota(jnp.int32, 16) == 0, 1.0, 0.0)
        e_bas = jnp.where(lax.iota(jnp.int32, 16) == 1, 1.0, 0.0)
        pltpu.sync_copy(zro_hbm.at[pl.ds(s * rz, rz)], acc.at[pl.ds(s * rz, rz)])
        plsc.subcore_barrier()

        def chunk_body(kk, _):
            base = (wid * chunks + kk) * _B
            pltpu.sync_copy(dst_hbm.at[pl.ds(base, _B)], dst_v)
            pltpu.sync_copy(bas_hbm.at[pl.ds(base, _B)], bas_v)

            def grp_body(t, _):
                bvec = bas_v[pl.ds(16 * t, 16)]
                for i in range(16):
                    r = 16 * t + i
                    ind = jnp.where(base + r < e, 1.0, 0.0)
                    rows_v[r, :] = e_deg * ind + e_bas * bvec[i]
                return 0

            lax.fori_loop(0, _B // 16, grp_body, 0)
            pltpu.sync_copy(rows_v, acc.at[dst_v], add=True)
            return 0

        lax.fori_loop(0, chunks, chunk_body, 0)
        plsc.subcore_barrier()
        pltpu.sync_copy(acc.at[pl.ds(s * rz, rz)], out_hbm.at[c, pl.ds(s * rz, rz)])

    return k


def _deg_s(dst_p, bas, e, n_out):
    e_pad = bas.shape[0]
    n_pad = _ceil_to(n_out, 128)
    k = _make_deg_s(e, e_pad, n_pad)
    zro = jnp.zeros((n_pad, 16), jnp.float32)
    out = k(dst_p, bas, zro)
    g = (out[0] + out[1])[:n_out]
    return jnp.clip(g[:, 0], 1.0), g[:, 1]


def _spline(p, x_pad, src_p, dst_p, bas, deg, s=None):
    n, w = x_pad.shape
    in_dim = p['Wm'].shape[0]
    g = _edge_agg(x_pad, src_p, dst_p, bas, n)
    if in_dim == w + 1:
        agg = g @ p['Wm'][:w] + s[:, None] * p['Wm'][w][None, :]
        lin = x_pad @ p['Wr'][:w] + p['Wr'][w][None, :]
    else:
        agg = g[:, :in_dim] @ p['Wm']
        lin = x_pad[:, :in_dim] @ p['Wr']
    return agg / deg[:, None] + lin + p['b']


def _lin(p, x):
    return x @ p['W'] + p['b']


def _pool_max(x, cluster, n_out):
    out = jax.ops.segment_max(x, cluster, num_segments=n_out)
    return jnp.where(jnp.isfinite(out), out, 0.0)


def _pool_mean(x, cluster, n_out):
    s = jax.ops.segment_sum(x, cluster, num_segments=n_out)
    c = jax.ops.segment_sum(jnp.ones((x.shape[0],), jnp.float32), cluster, num_segments=n_out)
    return s / jnp.clip(c, 1.0)[:, None]


def kernel(x, pos, pseudo0, pseudo1, pseudo2, pseudo3, pseudo4, pseudo5, params, edge_index0, edge_index1, edge_index2, edge_index3, edge_index4, edge_index5, cluster1, cluster2, cluster3, cluster4, cluster5):
    edges = [edge_index0, edge_index1, edge_index2, edge_index3, edge_index4, edge_index5]
    pseudos = [pseudo0, pseudo1, pseudo2, pseudo3, pseudo4, pseudo5]
    clusters = [cluster1, cluster2, cluster3, cluster4, cluster5]
    elu = jax.nn.elu

    srcs, dsts, bass, degs, ss = [], [], [], [], []
    for l in range(6):
        e = edges[l].shape[1]
        e_pad = _ceil_to(e, _NC * _NS * _B)
        srcs.append(jnp.pad(edges[l][0], (0, e_pad - e)))
        dsts.append(jnp.pad(edges[l][1], (0, e_pad - e)))
        bass.append(_basis(pseudos[l], e_pad))
        deg, s = _deg_s(dsts[l], bass[l], e, _N_LVL[l])
        degs.append(deg)
        ss.append(s)

    x16 = jnp.pad(x, ((0, 0), (0, 15)))
    h = _spline(params['conv1'], x16, srcs[0], dsts[0], bass[0], degs[0])
    x0 = elu(h)
    x1p = _pool_max(x0, clusters[0], _N_LVL[1])

    h = _spline(params['conv2'], x1p, srcs[1], dsts[1], bass[1], degs[1], ss[1])
    h = _spline(params['conv22'], elu(h), srcs[1], dsts[1], bass[1], degs[1])
    x1 = elu(h + _lin(params['skip1'], x1p))
    x2p = _pool_max(x1, clusters[1], _N_LVL[2])

    h = _spline(params['conv3'], x2p, srcs[2], dsts[2], bass[2], degs[2], ss[2])
    h = _spline(params['conv32'], elu(h), srcs[2], dsts[2], bass[2], degs[2])
    x2 = elu(h + x2p)
    x3p = _pool_max(x2, clusters[2], _N_LVL[3])

    h = _spline(params['conv4'], x3p, srcs[3], dsts[3], bass[3], degs[3], ss[3])
    h = _spline(params['conv42'], elu(h), srcs[3], dsts[3], bass[3], degs[3])
    x3 = elu(h + x3p)
    x4p = _pool_max(x3, clusters[3], _N_LVL[4])

    h = _spline(params['conv5'], x4p, srcs[4], dsts[4], bass[4], degs[4], ss[4])
    h = _spline(params['conv52'], elu(h), srcs[4], dsts[4], bass[4], degs[4])
    x4 = elu(h + _lin(params['skip2'], x4p))
    x5p = _pool_max(x4, clusters[4], _N_LVL[5])

    h = _spline(params['conv6'], x5p, srcs[5], dsts[5], bass[5], degs[5], ss[5])
    h = _spline(params['conv62'], elu(h), srcs[5], dsts[5], bass[5], degs[5])
    x5 = elu(h + _lin(params['skip3'], x5p))
    x5 = _lin(params['fc1'], x5)

    up = jnp.take(jnp.take(x5, clusters[4], axis=0), clusters[3], axis=0)
    cat = jnp.concatenate([up, jnp.take(x4, clusters[3], axis=0), _lin(params['skip_out'], x3)], axis=1)
    r = elu(_spline(params['convRPN1'], cat, srcs[3], dsts[3], bass[3], degs[3]))
    r = elu(_spline(params['convRPN2'], r, srcs[3], dsts[3], bass[3], degs[3]))
    r = _spline(params['convRPN3'], r, srcs[3], dsts[3], bass[3], degs[3])

    pos_c = pos
    for l in range(1, 6):
        pos_c = _pool_mean(pos_c, clusters[l - 1], _N_LVL[l])
    return (jax.nn.log_softmax(r[:, :2], axis=1), r[:, 2:], pos_c)
